# initial kernel scaffold (unmeasured)
import jax
import jax.numpy as jnp
from jax import lax
from jax.experimental import pallas as pl
from jax.experimental.pallas import tpu as pltpu

S_HALF = 1024
K = 4096
N = 8192
TN = 256
T = N // TN

DOT_DIMS = (((1,), (0,)), ((), ()))


def kernel(O, Wo):
    A = O.reshape(2 * S_HALF, K).astype(jnp.bfloat16)

    def body(a_ref, w_ref, out_ref, send_buf, recv_buf, send_sems, recv_sems):
        t = pl.program_id(0)
        p = pl.program_id(1)
        my_x = lax.axis_index("x")
        my_y = lax.axis_index("y")
        nbr = (1 - my_x, my_y)

        @pl.when(jnp.logical_and(t == 0, p == 0))
        def _():
            bar = pltpu.get_barrier_semaphore()
            pl.semaphore_signal(
                bar, inc=1, device_id=nbr, device_id_type=pl.DeviceIdType.MESH
            )
            pl.semaphore_wait(bar, 1)

        slot = lax.rem(t, 2)
        w_bf = w_ref[...].astype(jnp.bfloat16)

        def make_rdma():
            return pltpu.make_async_remote_copy(
                src_ref=send_buf.at[slot],
                dst_ref=recv_buf.at[t],
                send_sem=send_sems.at[t],
                recv_sem=recv_sems.at[t],
                device_id=nbr,
                device_id_type=pl.DeviceIdType.MESH,
            )

        @pl.when(p == 0)
        def _():
            r_nbr = (1 - my_x) * S_HALF
            e = lax.dot_general(
                a_ref[pl.ds(r_nbr, S_HALF), :], w_bf, DOT_DIMS,
                preferred_element_type=jnp.float32,
            )
            send_buf[slot] = e.astype(jnp.bfloat16)
            make_rdma().start()

        @pl.when(p == 1)
        def _():
            r_own = my_x * S_HALF
            l = lax.dot_general(
                a_ref[pl.ds(r_own, S_HALF), :], w_bf, DOT_DIMS,
                preferred_element_type=jnp.float32,
            )
            rdma = make_rdma()
            rdma.wait_recv()
            out_ref[...] = l + recv_buf[t].astype(jnp.float32)
            rdma.wait_send()

    out = pl.pallas_call(
        body,
        grid=(T, 2),
        out_shape=jax.ShapeDtypeStruct((S_HALF, N), jnp.float32),
        in_specs=[
            pl.BlockSpec((2 * S_HALF, K), lambda t, p: (0, 0),
                         memory_space=pltpu.VMEM),
            pl.BlockSpec((K, TN), lambda t, p: (0, t)),
        ],
        out_specs=pl.BlockSpec((S_HALF, TN), lambda t, p: (0, t)),
        scratch_shapes=[
            pltpu.VMEM((2, S_HALF, TN), jnp.bfloat16),
            pltpu.VMEM((T, S_HALF, TN), jnp.bfloat16),
            pltpu.SemaphoreType.DMA((T,)),
            pltpu.SemaphoreType.DMA((T,)),
        ],
        compiler_params=pltpu.CompilerParams(
            collective_id=0,
            dimension_semantics=("arbitrary", "arbitrary"),
        ),
    )(A, Wo)
    return out.reshape(1, S_HALF, N)


# baseline (device time: 461609 ns/iter reference)
import jax
import jax.numpy as jnp
from jax import lax
from jax.experimental import pallas as pl
from jax.experimental.pallas import tpu as pltpu

S_HALF = 1024
K = 4096
N = 8192
TN = 256
T = N // TN

DOT_DIMS = (((1,), (0,)), ((), ()))


def kernel(O, Wo):
    A = O.reshape(2 * S_HALF, K).astype(jnp.bfloat16)

    def body(a_ref, w_ref, out_ref, send_buf, recv_buf, send_sems, recv_sems):
        t = pl.program_id(0)
        p = pl.program_id(1)
        my_x = lax.axis_index("x")
        my_y = lax.axis_index("y")
        nbr = (1 - my_x, my_y)

        @pl.when(jnp.logical_and(t == 0, p == 0))
        def _():
            bar = pltpu.get_barrier_semaphore()
            pl.semaphore_signal(
                bar, inc=1, device_id=nbr, device_id_type=pl.DeviceIdType.MESH
            )
            pl.semaphore_wait(bar, 1)

        slot = lax.rem(t, 2)
        w_bf = w_ref[...].astype(jnp.bfloat16)

        def make_rdma():
            return pltpu.make_async_remote_copy(
                src_ref=send_buf.at[slot],
                dst_ref=recv_buf.at[t],
                send_sem=send_sems.at[t],
                recv_sem=recv_sems.at[t],
                device_id=nbr,
                device_id_type=pl.DeviceIdType.MESH,
            )

        @pl.when(p == 0)
        def _():
            r_nbr = (1 - my_x) * S_HALF
            e = lax.dot_general(
                a_ref[pl.ds(r_nbr, S_HALF), :], w_bf, DOT_DIMS,
                preferred_element_type=jnp.float32,
            )
            send_buf[slot] = e.astype(jnp.bfloat16)
            make_rdma().start()

        @pl.when(p == 1)
        def _():
            r_own = my_x * S_HALF
            l = lax.dot_general(
                a_ref[pl.ds(r_own, S_HALF), :], w_bf, DOT_DIMS,
                preferred_element_type=jnp.float32,
            )
            rdma = make_rdma()
            rdma.wait_recv()
            out_ref[...] = l + recv_buf[t].astype(jnp.float32)
            rdma.wait_send()

    out = pl.pallas_call(
        body,
        grid=(T, 2),
        out_shape=jax.ShapeDtypeStruct((S_HALF, N), jnp.float32),
        in_specs=[
            pl.BlockSpec((2 * S_HALF, K), lambda t, p: (0, 0),
                         memory_space=pltpu.VMEM),
            pl.BlockSpec((K, TN), lambda t, p: (0, t)),
        ],
        out_specs=pl.BlockSpec((S_HALF, TN), lambda t, p: (0, t)),
        scratch_shapes=[
            pltpu.VMEM((2, S_HALF, TN), jnp.bfloat16),
            pltpu.VMEM((T, S_HALF, TN), jnp.bfloat16),
            pltpu.SemaphoreType.DMA((T,)),
            pltpu.SemaphoreType.DMA((T,)),
        ],
        compiler_params=pltpu.CompilerParams(
            collective_id=0,
            dimension_semantics=("arbitrary", "arbitrary"),
            vmem_limit_bytes=100 * 1024 * 1024,
        ),
    )(A, Wo)
    return out.reshape(1, S_HALF, N)


# device time: 271051 ns/iter; 1.7030x vs baseline; 1.7030x over previous
import jax
import jax.numpy as jnp
from jax import lax
from jax.experimental import pallas as pl
from jax.experimental.pallas import tpu as pltpu

S_HALF = 1024
K = 4096
N = 8192
NH = N // 2
TN = 256
T = NH // TN

DOT_DIMS = (((1,), (0,)), ((), ()))


def kernel(O, Wo):
    my_x = lax.axis_index("x")
    my_y = lax.axis_index("y")
    A = O.reshape(2 * S_HALF, K).astype(jnp.bfloat16)
    a_nbr = lax.dynamic_slice(A, ((1 - my_x) * S_HALF, 0), (S_HALF, K))
    a_own = lax.dynamic_slice(A, (my_x * S_HALF, 0), (S_HALF, K))
    bases = jnp.stack([my_y * T, (1 - my_y) * T]).astype(jnp.int32)

    def body(s_ref, an_ref, ao_ref, w_ref, out_ref,
             sendx, recvx, stash, sendy, recvy,
             sx_sems, rx_sems, sy_sems, ry_sems):
        t = pl.program_id(0)
        p = pl.program_id(1)
        x = lax.axis_index("x")
        y = lax.axis_index("y")
        xnbr = (1 - x, y)
        ynbr = (x, 1 - y)

        def mk_x(j):
            return pltpu.make_async_remote_copy(
                src_ref=sendx.at[lax.rem(j, 2)],
                dst_ref=recvx.at[j],
                send_sem=sx_sems.at[j],
                recv_sem=rx_sems.at[j],
                device_id=xnbr,
                device_id_type=pl.DeviceIdType.MESH,
            )

        def mk_y(j):
            return pltpu.make_async_remote_copy(
                src_ref=sendy.at[lax.rem(j, 2)],
                dst_ref=recvy.at[j],
                send_sem=sy_sems.at[j],
                recv_sem=ry_sems.at[j],
                device_id=ynbr,
                device_id_type=pl.DeviceIdType.MESH,
            )

        @pl.when(jnp.logical_and(t == 0, p == 0))
        def _():
            bar = pltpu.get_barrier_semaphore()
            for nbr in (xnbr, ynbr):
                pl.semaphore_signal(
                    bar, inc=1, device_id=nbr,
                    device_id_type=pl.DeviceIdType.MESH,
                )
            pl.semaphore_wait(bar, 2)

        @pl.when(p == 0)
        def _():
            @pl.when(jnp.logical_and(t >= 2, t <= T + 1))
            def _():
                mk_x(t - 2).wait_send()

            @pl.when(t < T)
            def _():
                w_bf = w_ref[...].astype(jnp.bfloat16)
                e = lax.dot_general(an_ref[...], w_bf, DOT_DIMS,
                                    preferred_element_type=jnp.float32)
                sendx[lax.rem(t, 2)] = e.astype(jnp.bfloat16)
                mk_x(t).start()
                stash[lax.rem(t, 2)] = lax.dot_general(
                    ao_ref[...], w_bf, DOT_DIMS,
                    preferred_element_type=jnp.float32)

            @pl.when(jnp.logical_and(t >= 3, t <= T + 2))
            def _():
                mk_y(t - 3).wait_send()

            @pl.when(jnp.logical_and(t >= 1, t <= T))
            def _():
                j = t - 1
                mk_x(j).wait_recv()
                ssum = stash[lax.rem(j, 2)] + recvx[j].astype(jnp.float32)
                out_ref[...] = ssum
                sendy[lax.rem(j, 2)] = ssum.astype(jnp.bfloat16)
                mk_y(j).start()

        @pl.when(jnp.logical_and(p == 1, t >= 3))
        def _():
            j = t - 3
            mk_y(j).wait_recv()
            out_ref[...] = recvy[j].astype(jnp.float32)

    def out_idx(t, p, s):
        in_x_phase = jnp.logical_and(p == 0, t <= T)
        idx = jnp.where(in_x_phase,
                        s[0] + jnp.clip(t - 1, 0, T - 1),
                        s[1] + jnp.clip(t - 3, 0, T - 1))
        return (0, idx)

    out = pl.pallas_call(
        body,
        grid_spec=pltpu.PrefetchScalarGridSpec(
            num_scalar_prefetch=1,
            grid=(T + 3, 2),
            in_specs=[
                pl.BlockSpec((S_HALF, K), lambda t, p, s: (0, 0),
                             memory_space=pltpu.VMEM),
                pl.BlockSpec((S_HALF, K), lambda t, p, s: (0, 0),
                             memory_space=pltpu.VMEM),
                pl.BlockSpec((K, TN),
                             lambda t, p, s: (0, s[0] + jnp.clip(t, 0, T - 1))),
            ],
            out_specs=pl.BlockSpec((S_HALF, TN), out_idx),
            scratch_shapes=[
                pltpu.VMEM((2, S_HALF, TN), jnp.bfloat16),
                pltpu.VMEM((T, S_HALF, TN), jnp.bfloat16),
                pltpu.VMEM((2, S_HALF, TN), jnp.float32),
                pltpu.VMEM((2, S_HALF, TN), jnp.bfloat16),
                pltpu.VMEM((T, S_HALF, TN), jnp.bfloat16),
                pltpu.SemaphoreType.DMA((T,)),
                pltpu.SemaphoreType.DMA((T,)),
                pltpu.SemaphoreType.DMA((T,)),
                pltpu.SemaphoreType.DMA((T,)),
            ],
        ),
        out_shape=jax.ShapeDtypeStruct((S_HALF, N), jnp.float32),
        compiler_params=pltpu.CompilerParams(
            collective_id=0,
            dimension_semantics=("arbitrary", "arbitrary"),
            vmem_limit_bytes=100 * 1024 * 1024,
        ),
    )(bases, a_nbr, a_own, Wo)
    return out.reshape(1, S_HALF, N)


# device time: 225432 ns/iter; 2.0477x vs baseline; 1.2024x over previous
import jax
import jax.numpy as jnp
from jax import lax
from jax.experimental import pallas as pl
from jax.experimental.pallas import tpu as pltpu

S_HALF = 1024
K = 4096
N = 8192
NH = N // 2
TN = 512
T = NH // TN

DOT_DIMS = (((1,), (0,)), ((), ()))


def kernel(O, Wo):
    my_x = lax.axis_index("x")
    my_y = lax.axis_index("y")
    A = O.reshape(2 * S_HALF, K).astype(jnp.bfloat16)
    a_nbr = lax.dynamic_slice(A, ((1 - my_x) * S_HALF, 0), (S_HALF, K))
    a_own = lax.dynamic_slice(A, (my_x * S_HALF, 0), (S_HALF, K))
    w_half = lax.dynamic_slice(Wo, (0, my_y * NH), (K, NH)).astype(jnp.bfloat16)
    bases = jnp.stack([my_y * T, (1 - my_y) * T]).astype(jnp.int32)

    def body(s_ref, an_ref, ao_ref, w_ref, out_ref,
             sendx, recvx, stash, sendy, recvy,
             sx_sems, rx_sems, sy_sems, ry_sems):
        t = pl.program_id(0)
        p = pl.program_id(1)
        x = lax.axis_index("x")
        y = lax.axis_index("y")
        xnbr = (1 - x, y)
        ynbr = (x, 1 - y)

        def mk_x(j):
            return pltpu.make_async_remote_copy(
                src_ref=sendx.at[lax.rem(j, 2)],
                dst_ref=recvx.at[j],
                send_sem=sx_sems.at[j],
                recv_sem=rx_sems.at[j],
                device_id=xnbr,
                device_id_type=pl.DeviceIdType.MESH,
            )

        def mk_y(j):
            return pltpu.make_async_remote_copy(
                src_ref=sendy.at[lax.rem(j, 2)],
                dst_ref=recvy.at[j],
                send_sem=sy_sems.at[j],
                recv_sem=ry_sems.at[j],
                device_id=ynbr,
                device_id_type=pl.DeviceIdType.MESH,
            )

        @pl.when(jnp.logical_and(t == 0, p == 0))
        def _():
            bar = pltpu.get_barrier_semaphore()
            for nbr in (xnbr, ynbr):
                pl.semaphore_signal(
                    bar, inc=1, device_id=nbr,
                    device_id_type=pl.DeviceIdType.MESH,
                )
            pl.semaphore_wait(bar, 2)

        @pl.when(p == 0)
        def _():
            @pl.when(jnp.logical_and(t >= 2, t <= T + 1))
            def _():
                mk_x(t - 2).wait_send()

            @pl.when(t < T)
            def _():
                w_bf = w_ref[...]
                e = lax.dot_general(an_ref[...], w_bf, DOT_DIMS,
                                    preferred_element_type=jnp.float32)
                sendx[lax.rem(t, 2)] = e.astype(jnp.bfloat16)
                mk_x(t).start()
                stash[lax.rem(t, 2)] = lax.dot_general(
                    ao_ref[...], w_bf, DOT_DIMS,
                    preferred_element_type=jnp.float32,
                ).astype(jnp.bfloat16)

            @pl.when(jnp.logical_and(t >= 3, t <= T + 2))
            def _():
                mk_y(t - 3).wait_send()

            @pl.when(jnp.logical_and(t >= 1, t <= T))
            def _():
                j = t - 1
                mk_x(j).wait_recv()
                ssum = (stash[lax.rem(j, 2)].astype(jnp.float32)
                        + recvx[j].astype(jnp.float32))
                out_ref[...] = ssum
                sendy[lax.rem(j, 2)] = ssum.astype(jnp.bfloat16)
                mk_y(j).start()

        @pl.when(jnp.logical_and(p == 1, t >= 3))
        def _():
            j = t - 3
            mk_y(j).wait_recv()
            out_ref[...] = recvy[j].astype(jnp.float32)

    def out_idx(t, p, s):
        in_x_phase = jnp.logical_and(p == 0, t <= T)
        idx = jnp.where(in_x_phase,
                        s[0] + jnp.clip(t - 1, 0, T - 1),
                        s[1] + jnp.clip(t - 3, 0, T - 1))
        return (0, idx)

    out = pl.pallas_call(
        body,
        grid_spec=pltpu.PrefetchScalarGridSpec(
            num_scalar_prefetch=1,
            grid=(T + 3, 2),
            in_specs=[
                pl.BlockSpec((S_HALF, K), lambda t, p, s: (0, 0),
                             memory_space=pltpu.VMEM),
                pl.BlockSpec((S_HALF, K), lambda t, p, s: (0, 0),
                             memory_space=pltpu.VMEM),
                pl.BlockSpec((K, TN),
                             lambda t, p, s: (0, jnp.clip(t, 0, T - 1))),
            ],
            out_specs=pl.BlockSpec((S_HALF, TN), out_idx),
            scratch_shapes=[
                pltpu.VMEM((2, S_HALF, TN), jnp.bfloat16),
                pltpu.VMEM((T, S_HALF, TN), jnp.bfloat16),
                pltpu.VMEM((2, S_HALF, TN), jnp.bfloat16),
                pltpu.VMEM((2, S_HALF, TN), jnp.bfloat16),
                pltpu.VMEM((T, S_HALF, TN), jnp.bfloat16),
                pltpu.SemaphoreType.DMA((T,)),
                pltpu.SemaphoreType.DMA((T,)),
                pltpu.SemaphoreType.DMA((T,)),
                pltpu.SemaphoreType.DMA((T,)),
            ],
        ),
        out_shape=jax.ShapeDtypeStruct((S_HALF, N), jnp.float32),
        compiler_params=pltpu.CompilerParams(
            collective_id=0,
            dimension_semantics=("arbitrary", "arbitrary"),
            vmem_limit_bytes=100 * 1024 * 1024,
        ),
    )(bases, a_nbr, a_own, w_half)
    return out.reshape(1, S_HALF, N)


# device time: 202332 ns/iter; 2.2814x vs baseline; 1.1142x over previous
import jax
import jax.numpy as jnp
from jax import lax
from jax.experimental import pallas as pl
from jax.experimental.pallas import tpu as pltpu

S_HALF = 1024
K = 4096
N = 8192
NH = N // 2
TN = 512
T = NH // TN

DOT_DIMS = (((1,), (0,)), ((), ()))


def kernel(O, Wo):
    my_x = lax.axis_index("x")
    my_y = lax.axis_index("y")
    A = O.reshape(2 * S_HALF, K).astype(jnp.bfloat16)
    a_nbr = lax.dynamic_slice(A, ((1 - my_x) * S_HALF, 0), (S_HALF, K))
    a_own = lax.dynamic_slice(A, (my_x * S_HALF, 0), (S_HALF, K))
    bases = jnp.stack([my_y * T, (1 - my_y) * T]).astype(jnp.int32)

    def body(s_ref, an_ref, ao_ref, w_ref, out_ref,
             sendx, recvx, stash, sendy, recvy,
             sx_sems, rx_sems, sy_sems, ry_sems):
        t = pl.program_id(0)
        p = pl.program_id(1)
        x = lax.axis_index("x")
        y = lax.axis_index("y")
        xnbr = (1 - x, y)
        ynbr = (x, 1 - y)

        def mk_x(j):
            return pltpu.make_async_remote_copy(
                src_ref=sendx.at[lax.rem(j, 2)],
                dst_ref=recvx.at[j],
                send_sem=sx_sems.at[j],
                recv_sem=rx_sems.at[j],
                device_id=xnbr,
                device_id_type=pl.DeviceIdType.MESH,
            )

        def mk_y(j):
            return pltpu.make_async_remote_copy(
                src_ref=sendy.at[lax.rem(j, 2)],
                dst_ref=recvy.at[j],
                send_sem=sy_sems.at[j],
                recv_sem=ry_sems.at[j],
                device_id=ynbr,
                device_id_type=pl.DeviceIdType.MESH,
            )

        @pl.when(jnp.logical_and(t == 0, p == 0))
        def _():
            bar = pltpu.get_barrier_semaphore()
            for nbr in (xnbr, ynbr):
                pl.semaphore_signal(
                    bar, inc=1, device_id=nbr,
                    device_id_type=pl.DeviceIdType.MESH,
                )
            pl.semaphore_wait(bar, 2)

        @pl.when(p == 0)
        def _():
            @pl.when(jnp.logical_and(t >= 2, t <= T + 1))
            def _():
                mk_x(t - 2).wait_send()

            @pl.when(t < T)
            def _():
                w_bf = w_ref[...].astype(jnp.bfloat16)
                e = lax.dot_general(an_ref[...], w_bf, DOT_DIMS,
                                    preferred_element_type=jnp.float32)
                sendx[lax.rem(t, 2)] = e.astype(jnp.bfloat16)
                mk_x(t).start()
                stash[lax.rem(t, 2)] = lax.dot_general(
                    ao_ref[...], w_bf, DOT_DIMS,
                    preferred_element_type=jnp.float32,
                ).astype(jnp.bfloat16)

            @pl.when(jnp.logical_and(t >= 3, t <= T + 2))
            def _():
                mk_y(t - 3).wait_send()

            @pl.when(jnp.logical_and(t >= 1, t <= T))
            def _():
                j = t - 1
                mk_x(j).wait_recv()
                ssum = (stash[lax.rem(j, 2)].astype(jnp.float32)
                        + recvx[j].astype(jnp.float32))
                out_ref[...] = ssum
                sendy[lax.rem(j, 2)] = ssum.astype(jnp.bfloat16)
                mk_y(j).start()

        @pl.when(jnp.logical_and(p == 1, t >= 3))
        def _():
            j = t - 3
            mk_y(j).wait_recv()
            out_ref[...] = recvy[j].astype(jnp.float32)

    def out_idx(t, p, s):
        in_x_phase = jnp.logical_and(p == 0, t <= T)
        idx = jnp.where(in_x_phase,
                        s[0] + jnp.clip(t - 1, 0, T - 1),
                        s[1] + jnp.clip(t - 3, 0, T - 1))
        return (0, idx)

    out = pl.pallas_call(
        body,
        grid_spec=pltpu.PrefetchScalarGridSpec(
            num_scalar_prefetch=1,
            grid=(T + 3, 2),
            in_specs=[
                pl.BlockSpec((S_HALF, K), lambda t, p, s: (0, 0),
                             memory_space=pltpu.VMEM),
                pl.BlockSpec((S_HALF, K), lambda t, p, s: (0, 0),
                             memory_space=pltpu.VMEM),
                pl.BlockSpec((K, TN),
                             lambda t, p, s: (0, s[0] + jnp.clip(t, 0, T - 1))),
            ],
            out_specs=pl.BlockSpec((S_HALF, TN), out_idx),
            scratch_shapes=[
                pltpu.VMEM((2, S_HALF, TN), jnp.bfloat16),
                pltpu.VMEM((T, S_HALF, TN), jnp.bfloat16),
                pltpu.VMEM((2, S_HALF, TN), jnp.bfloat16),
                pltpu.VMEM((2, S_HALF, TN), jnp.bfloat16),
                pltpu.VMEM((T, S_HALF, TN), jnp.bfloat16),
                pltpu.SemaphoreType.DMA((T,)),
                pltpu.SemaphoreType.DMA((T,)),
                pltpu.SemaphoreType.DMA((T,)),
                pltpu.SemaphoreType.DMA((T,)),
            ],
        ),
        out_shape=jax.ShapeDtypeStruct((S_HALF, N), jnp.float32),
        compiler_params=pltpu.CompilerParams(
            collective_id=0,
            dimension_semantics=("arbitrary", "arbitrary"),
            vmem_limit_bytes=100 * 1024 * 1024,
        ),
    )(bases, a_nbr, a_own, Wo)
    return out.reshape(1, S_HALF, N)
